# baseline (speedup 1.0000x reference)
import functools
import math

import jax
import jax.numpy as jnp
from jax import lax
from jax.experimental import pallas as pl
from jax.experimental.pallas import tpu as pltpu
from jax.experimental.pallas import tpu_sc as plsc

_SMOOTHING = 0.1
_CONFIDENCE = 1.0 - _SMOOTHING

_NUM_CORES = 2
_NUM_SUBCORES = 16
_NW = _NUM_CORES * _NUM_SUBCORES
_LANES = 16

_TC_BLOCK_ROWS = 16000


def _sc_gather_partials(x_flat, idx):
    b = idx.shape[0]
    bpw = b // _NW
    mesh = plsc.VectorSubcoreMesh(core_axis_name="c", subcore_axis_name="s")

    @functools.partial(
        pl.kernel,
        mesh=mesh,
        out_type=jax.ShapeDtypeStruct((_NW * _LANES,), jnp.float32),
        scratch_types=[
            pltpu.VMEM((bpw,), jnp.int32),
            pltpu.VMEM((bpw,), jnp.float32),
            pltpu.VMEM((_LANES,), jnp.float32),
            pltpu.SemaphoreType.DMA,
        ],
    )
    def gather_kernel(x_hbm, idx_hbm, out_hbm, idx_v, vals_v, part_v, sem):
        wid = lax.axis_index("s") * _NUM_CORES + lax.axis_index("c")
        base = wid * bpw
        pltpu.sync_copy(idx_hbm.at[pl.ds(base, bpw)], idx_v)
        pltpu.async_copy(x_hbm.at[idx_v], vals_v, sem).wait()
        acc = vals_v[pl.ds(0, _LANES)]
        for k in range(1, bpw // _LANES):
            acc = acc + vals_v[pl.ds(k * _LANES, _LANES)]
        part_v[...] = acc
        pltpu.sync_copy(part_v, out_hbm.at[pl.ds(wid * _LANES, _LANES)])

    return gather_kernel(x_flat, idx)


def _tc_reduce_combine(part2d, x2d, k_total, s_coef, g_coef):
    rows, _ = x2d.shape
    grid = rows // _TC_BLOCK_ROWS

    def body(part_ref, x_ref, out_ref, acc_ref):
        step = pl.program_id(0)

        @pl.when(step == 0)
        def _init():
            acc_ref[0] = 0.0

        acc_ref[0] += jnp.sum(x_ref[...])

        @pl.when(step == grid - 1)
        def _finish():
            g = jnp.sum(part_ref[...])
            out_ref[0] = k_total - s_coef * acc_ref[0] - g_coef * g

    return pl.pallas_call(
        body,
        grid=(grid,),
        in_specs=[
            pl.BlockSpec(part2d.shape, lambda i: (0, 0)),
            pl.BlockSpec((_TC_BLOCK_ROWS, 128), lambda i: (i, 0)),
        ],
        out_specs=pl.BlockSpec(memory_space=pltpu.SMEM),
        out_shape=jax.ShapeDtypeStruct((1,), jnp.float32),
        scratch_shapes=[pltpu.SMEM((1,), jnp.float32)],
    )(part2d, x2d)


def kernel(x, target):
    n, v = x.shape
    s = _SMOOTHING / (v - 1)
    c = _CONFIDENCE
    k_total = n * ((v - 1) * s * math.log(s) + c * math.log(c))

    idx = jnp.arange(n, dtype=jnp.int32) * v + target.astype(jnp.int32)
    part = _sc_gather_partials(x.reshape(n * v), idx)

    x2d = x.reshape((n * v) // 128, 128)
    part2d = part.reshape(_NW * _LANES // 128, 128)
    out = _tc_reduce_combine(part2d, x2d, k_total, s, c - s)
    return out[0]


# speedup vs baseline: 2.7267x; 2.7267x over previous
import functools
import math

import jax
import jax.numpy as jnp
from jax import lax
from jax.experimental import pallas as pl
from jax.experimental.pallas import tpu as pltpu

_SMOOTHING = 0.1
_CONFIDENCE = 1.0 - _SMOOTHING

_BC = 4096


def _tc_loss(x, tgt2d, k_total, s_coef, g_coef):
    n, v = x.shape
    grid = pl.cdiv(v, _BC)

    def body(tgt_ref, x_ref, out_ref, acc_ref):
        step = pl.program_id(0)

        @pl.when(step == 0)
        def _init():
            acc_ref[0] = 0.0
            acc_ref[1] = 0.0

        blk = x_ref[...]
        col0 = step * _BC
        tloc = tgt_ref[...] - col0
        lane = lax.broadcasted_iota(jnp.int32, (n, _BC), 1)
        eq = lane == tloc
        acc_ref[1] += jnp.sum(jnp.where(eq, blk, 0.0))

        @pl.when(step < grid - 1)
        def _bulk():
            acc_ref[0] += jnp.sum(blk)

        @pl.when(step == grid - 1)
        def _tail():
            valid = (col0 + lane) < v
            acc_ref[0] += jnp.sum(jnp.where(valid, blk, 0.0))
            out_ref[0] = k_total - s_coef * acc_ref[0] - g_coef * acc_ref[1]

    return pl.pallas_call(
        body,
        grid=(grid,),
        in_specs=[
            pl.BlockSpec((n, 1), lambda i: (0, 0)),
            pl.BlockSpec((n, _BC), lambda i: (0, i)),
        ],
        out_specs=pl.BlockSpec(memory_space=pltpu.SMEM),
        out_shape=jax.ShapeDtypeStruct((1,), jnp.float32),
        scratch_shapes=[pltpu.SMEM((2,), jnp.float32)],
        compiler_params=pltpu.CompilerParams(
            dimension_semantics=("arbitrary",),
        ),
    )(tgt2d, x)


def kernel(x, target):
    n, v = x.shape
    s = _SMOOTHING / (v - 1)
    c = _CONFIDENCE
    k_total = n * ((v - 1) * s * math.log(s) + c * math.log(c))

    tgt2d = target.astype(jnp.int32).reshape(n, 1)
    out = _tc_loss(x, tgt2d, k_total, s, c - s)
    return out[0]


# speedup vs baseline: 3.1224x; 1.1451x over previous
import functools
import math

import jax
import jax.numpy as jnp
from jax import lax
from jax.experimental import pallas as pl
from jax.experimental.pallas import tpu as pltpu

_SMOOTHING = 0.1
_CONFIDENCE = 1.0 - _SMOOTHING

_BR = 32


def _tc_loss(x, tgt2d, k_total, s_coef, g_coef):
    n, v = x.shape
    grid = n // _BR
    gs_ratio = g_coef / s_coef

    def body(tgt_ref, x_ref, out_ref, acc_ref):
        step = pl.program_id(0)

        @pl.when(step == 0)
        def _init():
            acc_ref[0] = 0.0

        blk = x_ref[...]
        tloc = tgt_ref[...]
        lane = lax.broadcasted_iota(jnp.int32, (_BR, v), 1)
        gsum = jnp.sum(jnp.where(lane == tloc, blk, 0.0))
        acc_ref[0] += jnp.sum(blk) + gs_ratio * gsum

        @pl.when(step == grid - 1)
        def _fin():
            out_ref[0] = k_total - s_coef * acc_ref[0]

    return pl.pallas_call(
        body,
        grid=(grid,),
        in_specs=[
            pl.BlockSpec((_BR, 1), lambda i: (i, 0)),
            pl.BlockSpec((_BR, v), lambda i: (i, 0)),
        ],
        out_specs=pl.BlockSpec(memory_space=pltpu.SMEM),
        out_shape=jax.ShapeDtypeStruct((1,), jnp.float32),
        scratch_shapes=[pltpu.SMEM((1,), jnp.float32)],
        compiler_params=pltpu.CompilerParams(
            dimension_semantics=("arbitrary",),
        ),
    )(tgt2d, x)


def kernel(x, target):
    n, v = x.shape
    s = _SMOOTHING / (v - 1)
    c = _CONFIDENCE
    k_total = n * ((v - 1) * s * math.log(s) + c * math.log(c))

    tgt2d = target.astype(jnp.int32).reshape(n, 1)
    out = _tc_loss(x, tgt2d, k_total, s, c - s)
    return out[0]


# speedup vs baseline: 3.3234x; 1.0644x over previous
import functools
import math

import jax
import jax.numpy as jnp
from jax import lax
from jax.experimental import pallas as pl
from jax.experimental.pallas import tpu as pltpu

_SMOOTHING = 0.1
_CONFIDENCE = 1.0 - _SMOOTHING

_BR = 8
_NSTREAM = 4


def _tc_loss(x, tgt2d, k_total, s_coef, g_coef):
    n, v = x.shape
    rows_per_step = _BR * _NSTREAM
    grid = n // rows_per_step
    gs_ratio = g_coef / s_coef

    def body(tgt_ref, *rest):
        x_refs = rest[:_NSTREAM]
        out_ref, acc_ref = rest[_NSTREAM], rest[_NSTREAM + 1]
        step = pl.program_id(0)

        @pl.when(step == 0)
        def _init():
            acc_ref[0] = 0.0

        total = 0.0
        lane = lax.broadcasted_iota(jnp.int32, (_BR, v), 1)
        for k in range(_NSTREAM):
            blk = x_refs[k][...]
            tloc = tgt_ref[pl.ds(k * _BR, _BR), :]
            gsum = jnp.sum(jnp.where(lane == tloc, blk, 0.0))
            total += jnp.sum(blk) + gs_ratio * gsum
        acc_ref[0] += total

        @pl.when(step == grid - 1)
        def _fin():
            out_ref[0] = k_total - s_coef * acc_ref[0]

    def _mk_index_map(k):
        return lambda i: (i * _NSTREAM + k, 0)

    return pl.pallas_call(
        body,
        grid=(grid,),
        in_specs=[pl.BlockSpec((rows_per_step, 1), lambda i: (i, 0))]
        + [pl.BlockSpec((_BR, v), _mk_index_map(k)) for k in range(_NSTREAM)],
        out_specs=pl.BlockSpec(memory_space=pltpu.SMEM),
        out_shape=jax.ShapeDtypeStruct((1,), jnp.float32),
        scratch_shapes=[pltpu.SMEM((1,), jnp.float32)],
        compiler_params=pltpu.CompilerParams(
            dimension_semantics=("arbitrary",),
        ),
    )(tgt2d, *([x] * _NSTREAM))


def kernel(x, target):
    n, v = x.shape
    s = _SMOOTHING / (v - 1)
    c = _CONFIDENCE
    k_total = n * ((v - 1) * s * math.log(s) + c * math.log(c))

    tgt2d = target.astype(jnp.int32).reshape(n, 1)
    out = _tc_loss(x, tgt2d, k_total, s, c - s)
    return out[0]
